# Initial kernel scaffold; baseline (speedup 1.0000x reference)
import jax
import jax.numpy as jnp
from jax.experimental import pallas as pl


def kernel(cat_indices, num_values, emb_tables, num_W, num_b, W1, b1, W2, b2):
    raise NotImplementedError("write your pallas kernel here")


# speedup vs baseline: 7.3334x; 7.3334x over previous
import functools

import jax
import jax.numpy as jnp
from jax import lax
from jax.experimental import pallas as pl
from jax.experimental.pallas import tpu as pltpu
from jax.experimental.pallas import tpu_sc as plsc

B = 16384
F_CAT = 26
F_NUM = 13
V = 100000
D = 8
H = 128

NC, NS = 2, 16
NW = NC * NS
N_LOOK = B * F_CAT
N_PER_W = N_LOOK // NW
LANES = 16


def _sc_gather_body(idx_hbm, table_hbm, out_hbm, idx_v, rows_v, sem):
    wid = lax.axis_index("s") * NC + lax.axis_index("c")
    base = wid * N_PER_W
    pltpu.sync_copy(idx_hbm.at[pl.ds(base, N_PER_W)], idx_v)

    def add_offsets(j, carry):
        sl = pl.ds(j * LANES, LANES)
        pos = base + j * LANES + lax.iota(jnp.int32, LANES)
        f = lax.rem(pos, F_CAT)
        idx_v[sl] = idx_v[sl] + f * V
        return carry

    lax.fori_loop(0, N_PER_W // LANES, add_offsets, 0)

    pltpu.async_copy(table_hbm.at[idx_v], rows_v, sem).wait()
    pltpu.sync_copy(rows_v, out_hbm.at[pl.ds(base, N_PER_W)])


def _sc_gather(idx_flat, table_flat):
    mesh = plsc.VectorSubcoreMesh(
        core_axis_name="c", subcore_axis_name="s",
        num_cores=NC, num_subcores=NS)
    return pl.kernel(
        _sc_gather_body,
        out_type=jax.ShapeDtypeStruct((N_LOOK, D), jnp.float32),
        mesh=mesh,
        scratch_types=[
            pltpu.VMEM((N_PER_W,), jnp.int32),
            pltpu.VMEM((N_PER_W, D), jnp.float32),
            pltpu.SemaphoreType.DMA,
        ],
        compiler_params=pltpu.CompilerParams(use_tc_tiling_on_sc=False),
    )(idx_flat, table_flat)


BLK = 1024


def _mlp_body(cat_ref, nv_ref, nw_ref, nb_ref, w1_ref, b1_ref, w2_ref,
              b2_ref, out_ref):
    catf = cat_ref[...]
    nv = nv_ref[...]
    fi = lax.broadcasted_iota(jnp.int32, (F_NUM, F_NUM * D), 0)
    ji = lax.broadcasted_iota(jnp.int32, (F_NUM, F_NUM * D), 1)
    expand = jnp.where(ji // D == fi, 1.0, 0.0)
    rep = jnp.dot(nv, expand, preferred_element_type=jnp.float32)
    numf = rep * nw_ref[...] + nb_ref[...]
    h = (jnp.dot(catf, w1_ref[0:F_CAT * D, :],
                 preferred_element_type=jnp.float32)
         + jnp.dot(numf, w1_ref[F_CAT * D:, :],
                   preferred_element_type=jnp.float32)
         + b1_ref[...])
    h = jnp.maximum(h, 0.0)
    o = jnp.dot(h, w2_ref[...], preferred_element_type=jnp.float32) + b2_ref[...]
    out_ref[...] = jax.nn.sigmoid(o)


def _mlp(cat_flat, num_values, nw, nb, W1, b1r, W2, b2r):
    grid = (B // BLK,)
    return pl.pallas_call(
        _mlp_body,
        grid=grid,
        in_specs=[
            pl.BlockSpec((BLK, F_CAT * D), lambda i: (i, 0)),
            pl.BlockSpec((BLK, F_NUM), lambda i: (i, 0)),
            pl.BlockSpec((1, F_NUM * D), lambda i: (0, 0)),
            pl.BlockSpec((1, F_NUM * D), lambda i: (0, 0)),
            pl.BlockSpec(((F_CAT + F_NUM) * D, H), lambda i: (0, 0)),
            pl.BlockSpec((1, H), lambda i: (0, 0)),
            pl.BlockSpec((H, 1), lambda i: (0, 0)),
            pl.BlockSpec((1, 1), lambda i: (0, 0)),
        ],
        out_specs=pl.BlockSpec((BLK, 1), lambda i: (i, 0)),
        out_shape=jax.ShapeDtypeStruct((B, 1), jnp.float32),
    )(cat_flat, num_values, nw, nb, W1, b1r, W2, b2r)


def kernel(cat_indices, num_values, emb_tables, num_W, num_b, W1, b1, W2, b2):
    idx_flat = cat_indices.reshape(N_LOOK).astype(jnp.int32)
    table_flat = emb_tables.reshape(F_CAT * V, D)
    cat_flat = _sc_gather(idx_flat, table_flat).reshape(B, F_CAT * D)
    nw = num_W.reshape(1, F_NUM * D)
    nb = num_b.reshape(1, F_NUM * D)
    return _mlp(cat_flat, num_values, nw, nb, W1,
                b1.reshape(1, H), W2, b2.reshape(1, 1))


# speedup vs baseline: 23.4265x; 3.1945x over previous
import functools

import jax
import jax.numpy as jnp
from jax import lax
from jax.experimental import pallas as pl
from jax.experimental.pallas import tpu as pltpu
from jax.experimental.pallas import tpu_sc as plsc

B = 16384
F_CAT = 26
F_NUM = 13
V = 100000
D = 8
H = 128

NC, NS = 2, 16
NW = NC * NS
N_LOOK = B * F_CAT
N_PER_W = N_LOOK // NW
LANES = 16


N_CHUNK = 4
C_LOOK = N_PER_W // N_CHUNK
C_WORDS = C_LOOK * D


def _sc_gather_body(idx_hbm, table_hbm, out_hbm, idx_v, widx_v, rows_v, sem):
    wid = lax.axis_index("s") * NC + lax.axis_index("c")
    base = wid * N_PER_W
    pltpu.sync_copy(idx_hbm.at[pl.ds(base, N_PER_W)], idx_v)

    for c in range(N_CHUNK):
        def windex(j, carry, c=c):
            lanes = lax.iota(jnp.int32, LANES)
            half = lax.div(lanes, D)
            doff = lax.rem(lanes, D) * V
            i_local = c * C_LOOK + 2 * j + half
            f = lax.rem(base + i_local, F_CAT)
            v = plsc.load_gather(idx_v, [i_local])
            widx_v[pl.ds(j * LANES, LANES)] = f * (D * V) + doff + v
            return carry

        lax.fori_loop(0, C_WORDS // LANES, windex, 0)
        pltpu.async_copy(table_hbm.at[widx_v], rows_v, sem).wait()
        pltpu.sync_copy(
            rows_v, out_hbm.at[pl.ds((base + c * C_LOOK) * D, C_WORDS)])


def _sc_gather(idx_flat, table_lin):
    mesh = plsc.VectorSubcoreMesh(
        core_axis_name="c", subcore_axis_name="s",
        num_cores=NC, num_subcores=NS)
    return pl.kernel(
        _sc_gather_body,
        out_type=jax.ShapeDtypeStruct((N_LOOK * D,), jnp.float32),
        mesh=mesh,
        scratch_types=[
            pltpu.VMEM((N_PER_W,), jnp.int32),
            pltpu.VMEM((C_WORDS,), jnp.int32),
            pltpu.VMEM((C_WORDS,), jnp.float32),
            pltpu.SemaphoreType.DMA,
        ],
        compiler_params=pltpu.CompilerParams(
            use_tc_tiling_on_sc=False, needs_layout_passes=False),
    )(idx_flat, table_lin)


BLK = 1024


def _mlp_body(cat_ref, nv_ref, nw_ref, nb_ref, w1_ref, b1_ref, w2_ref,
              b2_ref, out_ref):
    catf = cat_ref[...]
    nv = nv_ref[...]
    fi = lax.broadcasted_iota(jnp.int32, (F_NUM, F_NUM * D), 0)
    ji = lax.broadcasted_iota(jnp.int32, (F_NUM, F_NUM * D), 1)
    expand = jnp.where(ji // D == fi, 1.0, 0.0)
    rep = jnp.dot(nv, expand, preferred_element_type=jnp.float32)
    numf = rep * nw_ref[...] + nb_ref[...]
    h = (jnp.dot(catf, w1_ref[0:F_CAT * D, :],
                 preferred_element_type=jnp.float32)
         + jnp.dot(numf, w1_ref[F_CAT * D:, :],
                   preferred_element_type=jnp.float32)
         + b1_ref[...])
    h = jnp.maximum(h, 0.0)
    o = jnp.dot(h, w2_ref[...], preferred_element_type=jnp.float32) + b2_ref[...]
    out_ref[...] = jax.nn.sigmoid(o)


def _mlp(cat_flat, num_values, nw, nb, W1, b1r, W2, b2r):
    grid = (B // BLK,)
    return pl.pallas_call(
        _mlp_body,
        grid=grid,
        in_specs=[
            pl.BlockSpec((BLK, F_CAT * D), lambda i: (i, 0)),
            pl.BlockSpec((BLK, F_NUM), lambda i: (i, 0)),
            pl.BlockSpec((1, F_NUM * D), lambda i: (0, 0)),
            pl.BlockSpec((1, F_NUM * D), lambda i: (0, 0)),
            pl.BlockSpec(((F_CAT + F_NUM) * D, H), lambda i: (0, 0)),
            pl.BlockSpec((1, H), lambda i: (0, 0)),
            pl.BlockSpec((H, 1), lambda i: (0, 0)),
            pl.BlockSpec((1, 1), lambda i: (0, 0)),
        ],
        out_specs=pl.BlockSpec((BLK, 1), lambda i: (i, 0)),
        out_shape=jax.ShapeDtypeStruct((B, 1), jnp.float32),
    )(cat_flat, num_values, nw, nb, W1, b1r, W2, b2r)


def kernel(cat_indices, num_values, emb_tables, num_W, num_b, W1, b1, W2, b2):
    idx_flat = cat_indices.reshape(N_LOOK).astype(jnp.int32)
    table_lin = emb_tables.transpose(0, 2, 1).reshape(F_CAT * D * V)
    cat_flat = _sc_gather(idx_flat, table_lin).reshape(B, F_CAT * D)
    nw = num_W.reshape(1, F_NUM * D)
    nb = num_b.reshape(1, F_NUM * D)
    return _mlp(cat_flat, num_values, nw, nb, W1,
                b1.reshape(1, H), W2, b2.reshape(1, 1))


# speedup vs baseline: 25.4627x; 1.0869x over previous
import functools

import jax
import jax.numpy as jnp
from jax import lax
from jax.experimental import pallas as pl
from jax.experimental.pallas import tpu as pltpu
from jax.experimental.pallas import tpu_sc as plsc

B = 16384
F_CAT = 26
F_NUM = 13
V = 100000
D = 8
H = 128

NC, NS = 2, 16
NW = NC * NS
N_LOOK = B * F_CAT
N_PER_W = N_LOOK // NW
LANES = 16


W_PER_W = N_PER_W * D
N_CHUNK = 8
C_WORDS = W_PER_W // N_CHUNK

_MESH = plsc.VectorSubcoreMesh(
    core_axis_name="c", subcore_axis_name="s",
    num_cores=NC, num_subcores=NS)
_SC_PARAMS = pltpu.CompilerParams(
    use_tc_tiling_on_sc=False, needs_layout_passes=False)


def _windex_body(idx_hbm, widx_hbm, fv_v, widx_v):
    wid = lax.axis_index("s") * NC + lax.axis_index("c")
    base = wid * N_PER_W
    pltpu.sync_copy(idx_hbm.at[pl.ds(base, N_PER_W)], fv_v)

    def flatfv(j, carry):
        sl = pl.ds(j * LANES, LANES)
        pos = base + j * LANES + lax.iota(jnp.int32, LANES)
        f = lax.rem(pos, F_CAT)
        fv_v[sl] = fv_v[sl] + f * (D * V)
        return carry

    lax.fori_loop(0, N_PER_W // LANES, flatfv, 0)

    def expand(j, carry):
        lanes = lax.iota(jnp.int32, LANES)
        half = lax.div(lanes, D)
        doff = lax.rem(lanes, D) * V
        fv = plsc.load_gather(fv_v, [2 * j + half])
        widx_v[pl.ds(j * LANES, LANES)] = fv + doff
        return carry

    lax.fori_loop(0, W_PER_W // LANES, expand, 0)
    pltpu.sync_copy(widx_v, widx_hbm.at[pl.ds(base * D, W_PER_W)])


def _sc_windex(idx_flat):
    return pl.kernel(
        _windex_body,
        out_type=jax.ShapeDtypeStruct((N_LOOK * D,), jnp.int32),
        mesh=_MESH,
        scratch_types=[
            pltpu.VMEM((N_PER_W,), jnp.int32),
            pltpu.VMEM((W_PER_W,), jnp.int32),
        ],
        compiler_params=_SC_PARAMS,
    )(idx_flat)


def _stream_body(widx_hbm, table_hbm, out_hbm, widx_v, rows_v,
                 gsem, wsem0, wsem1):
    wid = lax.axis_index("s") * NC + lax.axis_index("c")
    wbase = wid * W_PER_W
    wsems = (wsem0, wsem1)
    wr = [None, None]
    pltpu.sync_copy(widx_hbm.at[pl.ds(wbase, C_WORDS)], widx_v.at[0])
    for c in range(N_CHUNK):
        cur = c & 1
        if wr[cur] is not None:
            wr[cur].wait()
        g = pltpu.async_copy(
            table_hbm.at[widx_v.at[cur]], rows_v.at[cur], gsem)
        if c + 1 < N_CHUNK:
            pltpu.sync_copy(
                widx_hbm.at[pl.ds(wbase + (c + 1) * C_WORDS, C_WORDS)],
                widx_v.at[1 - cur])
        g.wait()
        wr[cur] = pltpu.async_copy(
            rows_v.at[cur],
            out_hbm.at[pl.ds(wbase + c * C_WORDS, C_WORDS)], wsems[cur])
    wr[0].wait()
    wr[1].wait()


def _sc_gather(idx_flat, table_lin):
    widx = _sc_windex(idx_flat)
    return pl.kernel(
        _stream_body,
        out_type=jax.ShapeDtypeStruct((N_LOOK * D,), jnp.float32),
        mesh=_MESH,
        scratch_types=[
            pltpu.VMEM((2, C_WORDS), jnp.int32),
            pltpu.VMEM((2, C_WORDS), jnp.float32),
            pltpu.SemaphoreType.DMA,
            pltpu.SemaphoreType.DMA,
            pltpu.SemaphoreType.DMA,
        ],
        compiler_params=_SC_PARAMS,
    )(widx, table_lin)


BLK = 1024


def _mlp_body(cat_ref, nv_ref, nw_ref, nb_ref, w1_ref, b1_ref, w2_ref,
              b2_ref, out_ref):
    catf = cat_ref[...]
    nv = nv_ref[...]
    fi = lax.broadcasted_iota(jnp.int32, (F_NUM, F_NUM * D), 0)
    ji = lax.broadcasted_iota(jnp.int32, (F_NUM, F_NUM * D), 1)
    expand = jnp.where(ji // D == fi, 1.0, 0.0)
    rep = jnp.dot(nv, expand, preferred_element_type=jnp.float32)
    numf = rep * nw_ref[...] + nb_ref[...]
    h = (jnp.dot(catf, w1_ref[0:F_CAT * D, :],
                 preferred_element_type=jnp.float32)
         + jnp.dot(numf, w1_ref[F_CAT * D:, :],
                   preferred_element_type=jnp.float32)
         + b1_ref[...])
    h = jnp.maximum(h, 0.0)
    o = jnp.dot(h, w2_ref[...], preferred_element_type=jnp.float32) + b2_ref[...]
    out_ref[...] = jax.nn.sigmoid(o)


def _mlp(cat_flat, num_values, nw, nb, W1, b1r, W2, b2r):
    grid = (B // BLK,)
    return pl.pallas_call(
        _mlp_body,
        grid=grid,
        in_specs=[
            pl.BlockSpec((BLK, F_CAT * D), lambda i: (i, 0)),
            pl.BlockSpec((BLK, F_NUM), lambda i: (i, 0)),
            pl.BlockSpec((1, F_NUM * D), lambda i: (0, 0)),
            pl.BlockSpec((1, F_NUM * D), lambda i: (0, 0)),
            pl.BlockSpec(((F_CAT + F_NUM) * D, H), lambda i: (0, 0)),
            pl.BlockSpec((1, H), lambda i: (0, 0)),
            pl.BlockSpec((H, 1), lambda i: (0, 0)),
            pl.BlockSpec((1, 1), lambda i: (0, 0)),
        ],
        out_specs=pl.BlockSpec((BLK, 1), lambda i: (i, 0)),
        out_shape=jax.ShapeDtypeStruct((B, 1), jnp.float32),
    )(cat_flat, num_values, nw, nb, W1, b1r, W2, b2r)


def kernel(cat_indices, num_values, emb_tables, num_W, num_b, W1, b1, W2, b2):
    idx_flat = cat_indices.reshape(N_LOOK).astype(jnp.int32)
    table_lin = emb_tables.transpose(0, 2, 1).reshape(F_CAT * D * V)
    cat_flat = _sc_gather(idx_flat, table_lin).reshape(B, F_CAT * D)
    nw = num_W.reshape(1, F_NUM * D)
    nb = num_b.reshape(1, F_NUM * D)
    return _mlp(cat_flat, num_values, nw, nb, W1,
                b1.reshape(1, H), W2, b2.reshape(1, 1))
